# baseline (speedup 1.0000x reference)
import functools

import jax
import jax.numpy as jnp
from jax import lax
from jax.experimental import pallas as pl
from jax.experimental.pallas import tpu as pltpu
from jax.experimental.pallas import tpu_sc as plsc

N_ATOMS = 10000
N_DIR = 320000
N_UND = 160000
ATOM_DIM = 128
HALF = ATOM_DIM // 2
HIDDEN = 64

NW = 32
C = 64
NCHUNK = 158
E_W = NCHUNK * C
N_PAD = NW * E_W
N_ACC = 10112
ROWS_PER_TILE = N_ACC // 16

_mesh = plsc.VectorSubcoreMesh(core_axis_name="c", subcore_axis_name="s")
_HI = -65536


def _pack_halves(lo, hi):
    loi = lax.bitcast_convert_type(lo.astype(jnp.bfloat16), jnp.uint16
                                   ).astype(jnp.int32)
    hii = lax.bitcast_convert_type(hi.astype(jnp.bfloat16), jnp.uint16
                                   ).astype(jnp.int32)
    return jnp.bitwise_or(loi, jnp.left_shift(hii, 16))


def _ptables_body(af_ref, wcl_ref, wch_ref, wnl_ref, wnh_ref, pc_ref, pn_ref):
    af = af_ref[...]
    pc_ref[...] = _pack_halves(
        jnp.dot(af, wcl_ref[...], preferred_element_type=jnp.float32),
        jnp.dot(af, wch_ref[...], preferred_element_type=jnp.float32))
    pn_ref[...] = _pack_halves(
        jnp.dot(af, wnl_ref[...], preferred_element_type=jnp.float32),
        jnp.dot(af, wnh_ref[...], preferred_element_type=jnp.float32))


def _btable_body(bf_ref, wbl_ref, wbh_ref, bbl_ref, bbh_ref, out_ref):
    bf = bf_ref[...]
    out_ref[...] = _pack_halves(
        jnp.dot(bf, wbl_ref[...], preferred_element_type=jnp.float32)
        + bbl_ref[...],
        jnp.dot(bf, wbh_ref[...], preferred_element_type=jnp.float32)
        + bbh_ref[...])


def _gather_h_body(pctr, pnbr, btab, cent2, nbr2, und2, hc, hn, hb,
                   cia, nia, uia,
                   bc0, bn0, bb0, bc1, bn1, bb1,
                   g0, g1, s0, s1):
    wid = lax.axis_index("s") * 2 + lax.axis_index("c")
    base = wid * E_W
    pltpu.sync_copy(cent2.at[wid], cia)
    pltpu.sync_copy(nbr2.at[wid], nia)
    pltpu.sync_copy(und2.at[wid], uia)
    sets = ((bc0, bn0, bb0, g0, s0), (bc1, bn1, bb1, g1, s1))

    def fire(q, st):
        bc, bn, bb, g, _ = st
        pltpu.async_copy(pctr.at[cia.at[pl.ds(q * C, C)]], bc, g)
        pltpu.async_copy(pnbr.at[nia.at[pl.ds(q * C, C)]], bn, g)
        pltpu.async_copy(btab.at[uia.at[pl.ds(q * C, C)]], bb, g)

    def wait_gathers(q, st):
        bc, bn, bb, g, _ = st
        pltpu.make_async_copy(pctr.at[cia.at[pl.ds(q * C, C)]], bc, g).wait()
        pltpu.make_async_copy(pnbr.at[nia.at[pl.ds(q * C, C)]], bn, g).wait()
        pltpu.make_async_copy(btab.at[uia.at[pl.ds(q * C, C)]], bb, g).wait()

    def fire_stores(q, st):
        bc, bn, bb, _, s = st
        sl = pl.ds(base + q * C, C)
        pltpu.async_copy(bc, hc.at[sl], s)
        pltpu.async_copy(bn, hn.at[sl], s)
        pltpu.async_copy(bb, hb.at[sl], s)

    def wait_stores(q, st):
        bc, bn, bb, _, s = st
        sl = pl.ds(base + q * C, C)
        pltpu.make_async_copy(bc, hc.at[sl], s).wait()
        pltpu.make_async_copy(bn, hn.at[sl], s).wait()
        pltpu.make_async_copy(bb, hb.at[sl], s).wait()

    fire(0, sets[0])

    def body(k, carry):
        for b in (0, 1):
            q = 2 * k + b
            st = sets[b]

            @pl.when(q >= 1)
            def _():
                wait_stores(q - 1, sets[1 - b])

            @pl.when(q + 1 < NCHUNK)
            def _():
                fire(q + 1, sets[1 - b])

            wait_gathers(q, st)
            fire_stores(q, st)
        return carry

    lax.fori_loop(0, NCHUNK // 2, body, 0)
    wait_stores(NCHUNK - 1, sets[1])


_gather_h = functools.partial(
    pl.kernel,
    out_type=[
        jax.ShapeDtypeStruct((N_PAD, HALF), jnp.int32),
        jax.ShapeDtypeStruct((N_PAD, HALF), jnp.int32),
        jax.ShapeDtypeStruct((N_PAD, HALF), jnp.int32),
    ],
    mesh=_mesh,
    compiler_params=pltpu.CompilerParams(use_tc_tiling_on_sc=False),
    scratch_types=[
        pltpu.VMEM((E_W,), jnp.int32),
        pltpu.VMEM((E_W,), jnp.int32),
        pltpu.VMEM((E_W,), jnp.int32),
        pltpu.VMEM((C, HALF), jnp.int32),
        pltpu.VMEM((C, HALF), jnp.int32),
        pltpu.VMEM((C, HALF), jnp.int32),
        pltpu.VMEM((C, HALF), jnp.int32),
        pltpu.VMEM((C, HALF), jnp.int32),
        pltpu.VMEM((C, HALF), jnp.int32),
        pltpu.SemaphoreType.DMA,
        pltpu.SemaphoreType.DMA,
        pltpu.SemaphoreType.DMA,
        pltpu.SemaphoreType.DMA,
    ],
)(_gather_h_body)


def _mlp_body(hc_ref, hn_ref, hb_ref, w2lo_ref, w2hi_ref, b2_ref, o_ref):
    a = hc_ref[...]
    b = hn_ref[...]
    c = hb_ref[...]
    lo = (lax.bitcast_convert_type(jnp.left_shift(a, 16), jnp.float32)
          + lax.bitcast_convert_type(jnp.left_shift(b, 16), jnp.float32)
          + lax.bitcast_convert_type(jnp.left_shift(c, 16), jnp.float32))
    hi = (lax.bitcast_convert_type(jnp.bitwise_and(a, _HI), jnp.float32)
          + lax.bitcast_convert_type(jnp.bitwise_and(b, _HI), jnp.float32)
          + lax.bitcast_convert_type(jnp.bitwise_and(c, _HI), jnp.float32))
    slo = lo * jax.nn.sigmoid(lo)
    shi = hi * jax.nn.sigmoid(hi)
    t = (jnp.dot(slo, w2lo_ref[...], preferred_element_type=jnp.float32)
         + jnp.dot(shi, w2hi_ref[...], preferred_element_type=jnp.float32)
         + b2_ref[...])
    core = t[:, :ATOM_DIM]
    gate = t[:, ATOM_DIM:]
    o_ref[...] = core * jax.nn.sigmoid(core) * jax.nn.sigmoid(gate)


def _scatter_body(msg, bwt, und2, cent2, zeros, out,
                  uia, cia, m0, w0, m1, w1, acc,
                  g0, g1, sc0, sc1, ci0, ci1):
    cid = lax.axis_index("c")
    sid = lax.axis_index("s")
    wid = sid * 2 + cid
    r0 = sid * ROWS_PER_TILE
    pltpu.sync_copy(zeros.at[pl.ds(r0, ROWS_PER_TILE)],
                    acc.at[pl.ds(r0, ROWS_PER_TILE)])
    plsc.subcore_barrier()
    base = wid * E_W
    pltpu.sync_copy(und2.at[wid], uia)
    pltpu.sync_copy(cent2.at[wid, pl.ds(0, C)], cia.at[0])
    sets = ((m0, w0, g0, sc0, ci0), (m1, w1, g1, sc1, ci1))

    def fire(q, st):
        m, w, g, _, _ = st
        pltpu.async_copy(msg.at[pl.ds(base + q * C, C)], m, g)
        pltpu.async_copy(bwt.at[uia.at[pl.ds(q * C, C)]], w, g)

    def wait_gathers(q, st):
        m, w, g, _, _ = st
        pltpu.make_async_copy(msg.at[pl.ds(base + q * C, C)], m, g).wait()
        pltpu.make_async_copy(bwt.at[uia.at[pl.ds(q * C, C)]], w, g).wait()

    fire(0, sets[0])

    def body(k, carry):
        for b in (0, 1):
            q = 2 * k + b
            m, w, g, sc, ci = sets[b]
            mo, wo, go, sco, cio = sets[1 - b]

            @pl.when(q >= 1)
            def _():
                pltpu.make_async_copy(mo, acc.at[cia.at[1 - b]], sco).wait()

            @pl.when(q + 1 < NCHUNK)
            def _():
                fire(q + 1, sets[1 - b])
                pltpu.async_copy(cent2.at[wid, pl.ds((q + 1) * C, C)],
                                 cia.at[1 - b], cio)

            wait_gathers(q, sets[b])

            def row(r, rc):
                for j in range(ATOM_DIM // 16):
                    sl = (r, pl.ds(j * 16, 16))
                    m[sl] = m[sl] * w[sl]
                return rc

            lax.fori_loop(0, C, row, 0)

            @pl.when(q >= 1)
            def _():
                pltpu.make_async_copy(cent2.at[wid, pl.ds(q * C, C)],
                                      cia.at[b], ci).wait()

            pltpu.async_copy(m, acc.at[cia.at[b]], sc, add=True)
        return carry

    lax.fori_loop(0, NCHUNK // 2, body, 0)
    pltpu.make_async_copy(m1, acc.at[cia.at[1]], sc1).wait()
    plsc.subcore_barrier()
    pltpu.sync_copy(acc.at[pl.ds(r0, ROWS_PER_TILE)],
                    out.at[cid, pl.ds(r0, ROWS_PER_TILE)])


_scatter = functools.partial(
    pl.kernel,
    out_type=jax.ShapeDtypeStruct((2, N_ACC, ATOM_DIM), jnp.float32),
    mesh=_mesh,
    scratch_types=[
        pltpu.VMEM((E_W,), jnp.int32),
        pltpu.VMEM((2, C), jnp.int32),
        pltpu.VMEM((C, ATOM_DIM), jnp.float32),
        pltpu.VMEM((C, ATOM_DIM), jnp.float32),
        pltpu.VMEM((C, ATOM_DIM), jnp.float32),
        pltpu.VMEM((C, ATOM_DIM), jnp.float32),
        pltpu.VMEM_SHARED((N_ACC, ATOM_DIM), jnp.float32),
        pltpu.SemaphoreType.DMA,
        pltpu.SemaphoreType.DMA,
        pltpu.SemaphoreType.DMA,
        pltpu.SemaphoreType.DMA,
        pltpu.SemaphoreType.DMA,
        pltpu.SemaphoreType.DMA,
    ],
)(_scatter_body)


def _final_body(p0_ref, p1_ref, wout_ref, bout_ref, af_ref, o_ref):
    a = p0_ref[...] + p1_ref[...]
    o_ref[...] = (
        jnp.dot(a, wout_ref[...], preferred_element_type=jnp.float32)
        + bout_ref[...]
        + af_ref[...]
    )


def kernel(atom_feas, bond_feas, bond_weights, atom_graph, directed2undirected,
           W1c, b1c, W2c, b2c, W1g, b1g, W2g, b2g, Wout, bout):
    f32 = jnp.float32
    Wctr = jnp.concatenate([W1c[:ATOM_DIM], W1g[:ATOM_DIM]], axis=1)
    Wnbr = jnp.concatenate([W1c[ATOM_DIM + 16:], W1g[ATOM_DIM + 16:]], axis=1)
    Wbnd = jnp.concatenate([W1c[ATOM_DIM:ATOM_DIM + 16],
                            W1g[ATOM_DIM:ATOM_DIM + 16]], axis=1)
    bcat1 = jnp.concatenate([b1c, b1g])[None, :]
    W2blk = jnp.zeros((ATOM_DIM, 2 * ATOM_DIM), f32)
    W2blk = W2blk.at[:HIDDEN, :ATOM_DIM].set(W2c)
    W2blk = W2blk.at[HIDDEN:, ATOM_DIM:].set(W2g)
    bcat2 = jnp.concatenate([b2c, b2g])[None, :]

    pad = N_PAD - N_DIR
    cent = jnp.concatenate(
        [atom_graph[:, 0], jnp.full((pad,), N_ATOMS, jnp.int32)])
    nbrs = jnp.concatenate([atom_graph[:, 1], jnp.zeros((pad,), jnp.int32)])
    und = jnp.concatenate([directed2undirected, jnp.zeros((pad,), jnp.int32)])
    cent2 = cent.reshape(NW, E_W)
    nbr2 = nbrs.reshape(NW, E_W)
    und2 = und.reshape(NW, E_W)
    af_pad = jnp.concatenate(
        [atom_feas, jnp.zeros((N_ACC - N_ATOMS, ATOM_DIM), f32)])

    pctr, pnbr = pl.pallas_call(
        _ptables_body,
        out_shape=[
            jax.ShapeDtypeStruct((N_ACC, HALF), jnp.int32),
            jax.ShapeDtypeStruct((N_ACC, HALF), jnp.int32),
        ],
    )(af_pad, Wctr[:, :HALF], Wctr[:, HALF:], Wnbr[:, :HALF], Wnbr[:, HALF:])

    btab = pl.pallas_call(
        _btable_body,
        grid=(20,),
        in_specs=[
            pl.BlockSpec((N_UND // 20, 16), lambda i: (i, 0)),
            pl.BlockSpec((16, HALF), lambda i: (0, 0)),
            pl.BlockSpec((16, HALF), lambda i: (0, 0)),
            pl.BlockSpec((1, HALF), lambda i: (0, 0)),
            pl.BlockSpec((1, HALF), lambda i: (0, 0)),
        ],
        out_specs=pl.BlockSpec((N_UND // 20, HALF), lambda i: (i, 0)),
        out_shape=jax.ShapeDtypeStruct((N_UND, HALF), jnp.int32),
    )(bond_feas, Wbnd[:, :HALF], Wbnd[:, HALF:],
      bcat1[:, :HALF], bcat1[:, HALF:])

    hc, hn, hb = _gather_h(pctr, pnbr, btab, cent2, nbr2, und2)

    BLK = 4096
    msg = pl.pallas_call(
        _mlp_body,
        grid=(N_PAD // BLK,),
        in_specs=[
            pl.BlockSpec((BLK, HALF), lambda i: (i, 0)),
            pl.BlockSpec((BLK, HALF), lambda i: (i, 0)),
            pl.BlockSpec((BLK, HALF), lambda i: (i, 0)),
            pl.BlockSpec((HALF, 2 * ATOM_DIM), lambda i: (0, 0)),
            pl.BlockSpec((HALF, 2 * ATOM_DIM), lambda i: (0, 0)),
            pl.BlockSpec((1, 2 * ATOM_DIM), lambda i: (0, 0)),
        ],
        out_specs=pl.BlockSpec((BLK, ATOM_DIM), lambda i: (i, 0)),
        out_shape=jax.ShapeDtypeStruct((N_PAD, ATOM_DIM), f32),
    )(hc, hn, hb, W2blk[:HALF], W2blk[HALF:], bcat2)

    zeros = jnp.zeros((N_ACC, ATOM_DIM), f32)
    partials = _scatter(msg, bond_weights, und2, cent2, zeros)

    out = pl.pallas_call(
        _final_body,
        out_shape=jax.ShapeDtypeStruct((N_ATOMS, ATOM_DIM), f32),
    )(partials[0, :N_ATOMS], partials[1, :N_ATOMS], Wout, bout[None, :],
      atom_feas)
    return out


# speedup vs baseline: 1.4994x; 1.4994x over previous
import functools

import jax
import jax.numpy as jnp
from jax import lax
from jax.experimental import pallas as pl
from jax.experimental.pallas import tpu as pltpu
from jax.experimental.pallas import tpu_sc as plsc

N_ATOMS = 10000
N_DIR = 320000
N_UND = 160000
ATOM_DIM = 128
HIDDEN = 64

NW = 32
C = 64
NCHUNK = 158
E_W = NCHUNK * C
N_PAD = NW * E_W
N_ACC = 10112
ROWS_PER_TILE = N_ACC // 16

_mesh = plsc.VectorSubcoreMesh(core_axis_name="c", subcore_axis_name="s")
_HI = -65536


def _ptables_body(af_ref, wctr_ref, wnbr_ref, pc_ref, pn_ref):
    af = af_ref[...]
    pc_ref[...] = jnp.dot(af, wctr_ref[...], preferred_element_type=jnp.float32)
    pn_ref[...] = jnp.dot(af, wnbr_ref[...], preferred_element_type=jnp.float32)


def _btable_body(bf_ref, bw_ref, wb_ref, bb_ref, out_ref):
    proj = (jnp.dot(bf_ref[...], wb_ref[...],
                    preferred_element_type=jnp.float32) + bb_ref[...])
    lo = lax.bitcast_convert_type(proj.astype(jnp.bfloat16), jnp.uint16
                                  ).astype(jnp.int32)
    hi = lax.bitcast_convert_type(bw_ref[...].astype(jnp.bfloat16), jnp.uint16
                                  ).astype(jnp.int32)
    out_ref[...] = jnp.bitwise_or(lo, jnp.left_shift(hi, 16))


def _gather_h_body(pctr, pnbr, btab, cent2, nbr2, und2, hp, bslab,
                   cia, nia, uia,
                   bc0, bn0, bb0, o0, bc1, bn1, bb1, o1,
                   g0, g1, s0, s1):
    wid = lax.axis_index("s") * 2 + lax.axis_index("c")
    base = wid * E_W
    pltpu.sync_copy(cent2.at[wid], cia)
    pltpu.sync_copy(nbr2.at[wid], nia)
    pltpu.sync_copy(und2.at[wid], uia)
    sets = ((bc0, bn0, bb0, o0, g0, s0), (bc1, bn1, bb1, o1, g1, s1))

    def fire(q, st):
        bc, bn, bb, _, g, _ = st
        pltpu.async_copy(pctr.at[cia.at[pl.ds(q * C, C)]], bc, g)
        pltpu.async_copy(pnbr.at[nia.at[pl.ds(q * C, C)]], bn, g)
        pltpu.async_copy(btab.at[uia.at[pl.ds(q * C, C)]], bb, g)

    def wait_gathers(q, st):
        bc, bn, bb, _, g, _ = st
        pltpu.make_async_copy(pctr.at[cia.at[pl.ds(q * C, C)]], bc, g).wait()
        pltpu.make_async_copy(pnbr.at[nia.at[pl.ds(q * C, C)]], bn, g).wait()
        pltpu.make_async_copy(btab.at[uia.at[pl.ds(q * C, C)]], bb, g).wait()

    def fire_stores(q, st):
        bc, bn, bb, o, _, s = st
        sl = pl.ds(base + q * C, C)
        pltpu.async_copy(o, hp.at[sl], s)
        pltpu.async_copy(bb, bslab.at[sl], s)

    def wait_stores(q, st):
        bc, bn, bb, o, _, s = st
        sl = pl.ds(base + q * C, C)
        pltpu.make_async_copy(o, hp.at[sl], s).wait()
        pltpu.make_async_copy(bb, bslab.at[sl], s).wait()

    fire(0, sets[0])

    def body(k, carry):
        for b in (0, 1):
            q = 2 * k + b
            st = sets[b]
            bc, bn, bb, o, g, s = st

            @pl.when(q >= 1)
            def _():
                wait_stores(q - 1, sets[1 - b])

            @pl.when(q + 1 < NCHUNK)
            def _():
                fire(q + 1, sets[1 - b])

            wait_gathers(q, st)

            def row(r, rc):
                for j in range(ATOM_DIM // 16):
                    sl = (r, pl.ds(j * 16, 16))
                    o[sl] = bc[sl] + bn[sl]
                return rc

            lax.fori_loop(0, C, row, 0)
            fire_stores(q, st)
        return carry

    lax.fori_loop(0, NCHUNK // 2, body, 0)
    wait_stores(NCHUNK - 1, sets[1])


_gather_h = functools.partial(
    pl.kernel,
    out_type=[
        jax.ShapeDtypeStruct((N_PAD, ATOM_DIM), jnp.float32),
        jax.ShapeDtypeStruct((N_PAD, ATOM_DIM), jnp.int32),
    ],
    mesh=_mesh,
    scratch_types=[
        pltpu.VMEM((E_W,), jnp.int32),
        pltpu.VMEM((E_W,), jnp.int32),
        pltpu.VMEM((E_W,), jnp.int32),
        pltpu.VMEM((C, ATOM_DIM), jnp.float32),
        pltpu.VMEM((C, ATOM_DIM), jnp.float32),
        pltpu.VMEM((C, ATOM_DIM), jnp.int32),
        pltpu.VMEM((C, ATOM_DIM), jnp.float32),
        pltpu.VMEM((C, ATOM_DIM), jnp.float32),
        pltpu.VMEM((C, ATOM_DIM), jnp.float32),
        pltpu.VMEM((C, ATOM_DIM), jnp.int32),
        pltpu.VMEM((C, ATOM_DIM), jnp.float32),
        pltpu.SemaphoreType.DMA,
        pltpu.SemaphoreType.DMA,
        pltpu.SemaphoreType.DMA,
        pltpu.SemaphoreType.DMA,
    ],
)(_gather_h_body)


def _mlp_body(hp_ref, bs_ref, w2_ref, b2_ref, o_ref):
    u = bs_ref[...]
    h = hp_ref[...] + lax.bitcast_convert_type(
        jnp.left_shift(u, 16), jnp.float32)
    bw = lax.bitcast_convert_type(jnp.bitwise_and(u, _HI), jnp.float32)
    s = h * jax.nn.sigmoid(h)
    t = jnp.dot(s, w2_ref[...], preferred_element_type=jnp.float32) + b2_ref[...]
    core = t[:, :ATOM_DIM]
    gate = t[:, ATOM_DIM:]
    o_ref[...] = core * jax.nn.sigmoid(core) * jax.nn.sigmoid(gate) * bw


def _scatter_body(msg, cent2, zeros, out,
                  cia, m0, m1, acc,
                  g0, g1, sc0, sc1, ci0, ci1):
    cid = lax.axis_index("c")
    sid = lax.axis_index("s")
    wid = sid * 2 + cid
    r0 = sid * ROWS_PER_TILE
    pltpu.sync_copy(zeros.at[pl.ds(r0, ROWS_PER_TILE)],
                    acc.at[pl.ds(r0, ROWS_PER_TILE)])
    plsc.subcore_barrier()
    base = wid * E_W
    pltpu.sync_copy(cent2.at[wid, pl.ds(0, C)], cia.at[0])
    sets = ((m0, g0, sc0, ci0), (m1, g1, sc1, ci1))

    pltpu.async_copy(msg.at[pl.ds(base, C)], m0, g0)

    def body(k, carry):
        for b in (0, 1):
            q = 2 * k + b
            m, g, sc, ci = sets[b]
            mo, go, sco, cio = sets[1 - b]

            @pl.when(q >= 1)
            def _():
                pltpu.make_async_copy(mo, acc.at[cia.at[1 - b]], sco).wait()

            @pl.when(q + 1 < NCHUNK)
            def _():
                pltpu.async_copy(msg.at[pl.ds(base + (q + 1) * C, C)], mo, go)
                pltpu.async_copy(cent2.at[wid, pl.ds((q + 1) * C, C)],
                                 cia.at[1 - b], cio)

            pltpu.make_async_copy(msg.at[pl.ds(base + q * C, C)], m, g).wait()

            @pl.when(q >= 1)
            def _():
                pltpu.make_async_copy(cent2.at[wid, pl.ds(q * C, C)],
                                      cia.at[b], ci).wait()

            pltpu.async_copy(m, acc.at[cia.at[b]], sc, add=True)
        return carry

    lax.fori_loop(0, NCHUNK // 2, body, 0)
    pltpu.make_async_copy(m1, acc.at[cia.at[1]], sc1).wait()
    plsc.subcore_barrier()
    pltpu.sync_copy(acc.at[pl.ds(r0, ROWS_PER_TILE)],
                    out.at[cid, pl.ds(r0, ROWS_PER_TILE)])


_scatter = functools.partial(
    pl.kernel,
    out_type=jax.ShapeDtypeStruct((2, N_ACC, ATOM_DIM), jnp.float32),
    mesh=_mesh,
    scratch_types=[
        pltpu.VMEM((2, C), jnp.int32),
        pltpu.VMEM((C, ATOM_DIM), jnp.float32),
        pltpu.VMEM((C, ATOM_DIM), jnp.float32),
        pltpu.VMEM_SHARED((N_ACC, ATOM_DIM), jnp.float32),
        pltpu.SemaphoreType.DMA,
        pltpu.SemaphoreType.DMA,
        pltpu.SemaphoreType.DMA,
        pltpu.SemaphoreType.DMA,
        pltpu.SemaphoreType.DMA,
        pltpu.SemaphoreType.DMA,
    ],
)(_scatter_body)


def _final_body(p0_ref, p1_ref, wout_ref, bout_ref, af_ref, o_ref):
    a = p0_ref[...] + p1_ref[...]
    o_ref[...] = (
        jnp.dot(a, wout_ref[...], preferred_element_type=jnp.float32)
        + bout_ref[...]
        + af_ref[...]
    )


def kernel(atom_feas, bond_feas, bond_weights, atom_graph, directed2undirected,
           W1c, b1c, W2c, b2c, W1g, b1g, W2g, b2g, Wout, bout):
    f32 = jnp.float32
    Wctr = jnp.concatenate([W1c[:ATOM_DIM], W1g[:ATOM_DIM]], axis=1)
    Wnbr = jnp.concatenate([W1c[ATOM_DIM + 16:], W1g[ATOM_DIM + 16:]], axis=1)
    Wbnd = jnp.concatenate([W1c[ATOM_DIM:ATOM_DIM + 16],
                            W1g[ATOM_DIM:ATOM_DIM + 16]], axis=1)
    bcat1 = jnp.concatenate([b1c, b1g])[None, :]
    W2blk = jnp.zeros((ATOM_DIM, 2 * ATOM_DIM), f32)
    W2blk = W2blk.at[:HIDDEN, :ATOM_DIM].set(W2c)
    W2blk = W2blk.at[HIDDEN:, ATOM_DIM:].set(W2g)
    bcat2 = jnp.concatenate([b2c, b2g])[None, :]

    pad = N_PAD - N_DIR
    cent = jnp.concatenate(
        [atom_graph[:, 0], jnp.full((pad,), N_ATOMS, jnp.int32)])
    nbrs = jnp.concatenate([atom_graph[:, 1], jnp.zeros((pad,), jnp.int32)])
    und = jnp.concatenate([directed2undirected, jnp.zeros((pad,), jnp.int32)])
    cent2 = cent.reshape(NW, E_W)
    nbr2 = nbrs.reshape(NW, E_W)
    und2 = und.reshape(NW, E_W)
    af_pad = jnp.concatenate(
        [atom_feas, jnp.zeros((N_ACC - N_ATOMS, ATOM_DIM), f32)])

    pctr, pnbr = pl.pallas_call(
        _ptables_body,
        out_shape=[
            jax.ShapeDtypeStruct((N_ACC, ATOM_DIM), f32),
            jax.ShapeDtypeStruct((N_ACC, ATOM_DIM), f32),
        ],
    )(af_pad, Wctr, Wnbr)

    btab = pl.pallas_call(
        _btable_body,
        grid=(20,),
        in_specs=[
            pl.BlockSpec((N_UND // 20, 16), lambda i: (i, 0)),
            pl.BlockSpec((N_UND // 20, ATOM_DIM), lambda i: (i, 0)),
            pl.BlockSpec((16, ATOM_DIM), lambda i: (0, 0)),
            pl.BlockSpec((1, ATOM_DIM), lambda i: (0, 0)),
        ],
        out_specs=pl.BlockSpec((N_UND // 20, ATOM_DIM), lambda i: (i, 0)),
        out_shape=jax.ShapeDtypeStruct((N_UND, ATOM_DIM), jnp.int32),
    )(bond_feas, bond_weights, Wbnd, bcat1)

    hp, bslab = _gather_h(pctr, pnbr, btab, cent2, nbr2, und2)

    BLK = 4096
    msg = pl.pallas_call(
        _mlp_body,
        grid=(N_PAD // BLK,),
        in_specs=[
            pl.BlockSpec((BLK, ATOM_DIM), lambda i: (i, 0)),
            pl.BlockSpec((BLK, ATOM_DIM), lambda i: (i, 0)),
            pl.BlockSpec((ATOM_DIM, 2 * ATOM_DIM), lambda i: (0, 0)),
            pl.BlockSpec((1, 2 * ATOM_DIM), lambda i: (0, 0)),
        ],
        out_specs=pl.BlockSpec((BLK, ATOM_DIM), lambda i: (i, 0)),
        out_shape=jax.ShapeDtypeStruct((N_PAD, ATOM_DIM), f32),
    )(hp, bslab, W2blk, bcat2)

    zeros = jnp.zeros((N_ACC, ATOM_DIM), f32)
    partials = _scatter(msg, cent2, zeros)

    out = pl.pallas_call(
        _final_body,
        out_shape=jax.ShapeDtypeStruct((N_ATOMS, ATOM_DIM), f32),
    )(partials[0, :N_ATOMS], partials[1, :N_ATOMS], Wout, bout[None, :],
      atom_feas)
    return out


# speedup vs baseline: 1.8488x; 1.2330x over previous
import functools

import jax
import jax.numpy as jnp
from jax import lax
from jax.experimental import pallas as pl
from jax.experimental.pallas import tpu as pltpu
from jax.experimental.pallas import tpu_sc as plsc

N_ATOMS = 10000
N_DIR = 320000
N_UND = 160000
ATOM_DIM = 128
HIDDEN = 64

NW = 32
C = 64
NCHUNK = 158
E_W = NCHUNK * C
N_PAD = NW * E_W
N_ACC = 10112
ROWS_PER_TILE = N_ACC // 16

_mesh = plsc.VectorSubcoreMesh(core_axis_name="c", subcore_axis_name="s")
_HI = -65536


def _ptables_body(af_ref, wctr_ref, wnbr_ref, pc_ref, pn_ref):
    af = af_ref[...]
    pc_ref[...] = jnp.dot(af, wctr_ref[...], preferred_element_type=jnp.float32)
    pn_ref[...] = jnp.dot(af, wnbr_ref[...], preferred_element_type=jnp.float32)


def _btable_body(bf_ref, bw_ref, wb_ref, bb_ref, out_ref):
    proj = (jnp.dot(bf_ref[...], wb_ref[...],
                    preferred_element_type=jnp.float32) + bb_ref[...])
    lo = lax.bitcast_convert_type(proj.astype(jnp.bfloat16), jnp.uint16
                                  ).astype(jnp.int32)
    hi = lax.bitcast_convert_type(bw_ref[...].astype(jnp.bfloat16), jnp.uint16
                                  ).astype(jnp.int32)
    out_ref[...] = jnp.bitwise_or(lo, jnp.left_shift(hi, 16))


def _gather_h_body(pctr, pnbr, btab, cent2, nbr2, und2, hp, bslab,
                   cia, nia, uia,
                   bc0, bn0, bb0, o0, bc1, bn1, bb1, o1,
                   g0, g1, s0, s1):
    wid = lax.axis_index("s") * 2 + lax.axis_index("c")
    base = wid * E_W
    pltpu.sync_copy(cent2.at[wid], cia)
    pltpu.sync_copy(nbr2.at[wid], nia)
    pltpu.sync_copy(und2.at[wid], uia)
    sets = ((bc0, bn0, bb0, o0, g0, s0), (bc1, bn1, bb1, o1, g1, s1))

    def fire(q, st):
        bc, bn, bb, _, g, _ = st
        pltpu.async_copy(pctr.at[cia.at[pl.ds(q * C, C)]], bc, g)
        pltpu.async_copy(pnbr.at[nia.at[pl.ds(q * C, C)]], bn, g)
        pltpu.async_copy(btab.at[uia.at[pl.ds(q * C, C)]], bb, g)

    def wait_gathers(q, st):
        bc, bn, bb, _, g, _ = st
        pltpu.make_async_copy(pctr.at[cia.at[pl.ds(q * C, C)]], bc, g).wait()
        pltpu.make_async_copy(pnbr.at[nia.at[pl.ds(q * C, C)]], bn, g).wait()
        pltpu.make_async_copy(btab.at[uia.at[pl.ds(q * C, C)]], bb, g).wait()

    def fire_stores(q, st):
        bc, bn, bb, o, _, s = st
        sl = pl.ds(base + q * C, C)
        pltpu.async_copy(o, hp.at[sl], s)
        pltpu.async_copy(bb, bslab.at[sl], s)

    def wait_stores(q, st):
        bc, bn, bb, o, _, s = st
        sl = pl.ds(base + q * C, C)
        pltpu.make_async_copy(o, hp.at[sl], s).wait()
        pltpu.make_async_copy(bb, bslab.at[sl], s).wait()

    fire(0, sets[0])

    def body(k, carry):
        for b in (0, 1):
            q = 2 * k + b
            st = sets[b]
            bc, bn, bb, o, g, s = st

            @pl.when(q >= 1)
            def _():
                wait_stores(q - 1, sets[1 - b])

            @pl.when(q + 1 < NCHUNK)
            def _():
                fire(q + 1, sets[1 - b])

            wait_gathers(q, st)

            def row(r, rc):
                for j in range(ATOM_DIM // 16):
                    sl = (r, pl.ds(j * 16, 16))
                    o[sl] = bc[sl] + bn[sl]
                return rc

            lax.fori_loop(0, C, row, 0)
            fire_stores(q, st)
        return carry

    lax.fori_loop(0, NCHUNK // 2, body, 0)
    wait_stores(NCHUNK - 1, sets[1])


_gather_h = functools.partial(
    pl.kernel,
    out_type=[
        jax.ShapeDtypeStruct((N_PAD, ATOM_DIM), jnp.float32),
        jax.ShapeDtypeStruct((N_PAD, ATOM_DIM), jnp.int32),
    ],
    mesh=_mesh,
    scratch_types=[
        pltpu.VMEM((E_W,), jnp.int32),
        pltpu.VMEM((E_W,), jnp.int32),
        pltpu.VMEM((E_W,), jnp.int32),
        pltpu.VMEM((C, ATOM_DIM), jnp.float32),
        pltpu.VMEM((C, ATOM_DIM), jnp.float32),
        pltpu.VMEM((C, ATOM_DIM), jnp.int32),
        pltpu.VMEM((C, ATOM_DIM), jnp.float32),
        pltpu.VMEM((C, ATOM_DIM), jnp.float32),
        pltpu.VMEM((C, ATOM_DIM), jnp.float32),
        pltpu.VMEM((C, ATOM_DIM), jnp.int32),
        pltpu.VMEM((C, ATOM_DIM), jnp.float32),
        pltpu.SemaphoreType.DMA,
        pltpu.SemaphoreType.DMA,
        pltpu.SemaphoreType.DMA,
        pltpu.SemaphoreType.DMA,
    ],
)(_gather_h_body)


def _mlp_body(hp_ref, bs_ref, w2_ref, b2_ref, o_ref):
    u = bs_ref[...]
    h = hp_ref[...] + lax.bitcast_convert_type(
        jnp.left_shift(u, 16), jnp.float32)
    bw = lax.bitcast_convert_type(jnp.bitwise_and(u, _HI), jnp.float32)
    s = h * jax.nn.sigmoid(h)
    t = jnp.dot(s, w2_ref[...], preferred_element_type=jnp.float32) + b2_ref[...]
    core = t[:, :ATOM_DIM]
    gate = t[:, ATOM_DIM:]
    o_ref[...] = core * jax.nn.sigmoid(core) * jax.nn.sigmoid(gate) * bw


def _scatter_body(msg, cent2, zeros, out,
                  cia, m0, m1, acc,
                  g0, g1, sc0, sc1, ci0, ci1):
    cid = lax.axis_index("c")
    sid = lax.axis_index("s")
    wid = sid * 2 + cid
    r0 = sid * ROWS_PER_TILE
    pltpu.sync_copy(zeros.at[pl.ds(r0, ROWS_PER_TILE)],
                    acc.at[pl.ds(r0, ROWS_PER_TILE)])
    plsc.subcore_barrier()
    base = wid * E_W
    pltpu.sync_copy(cent2.at[wid, pl.ds(0, C)], cia.at[0])
    sets = ((m0, g0, sc0, ci0), (m1, g1, sc1, ci1))

    pltpu.async_copy(msg.at[pl.ds(base, C)], m0, g0)

    def body(k, carry):
        for b in (0, 1):
            q = 2 * k + b
            m, g, sc, ci = sets[b]
            mo, go, sco, cio = sets[1 - b]

            @pl.when(q >= 1)
            def _():
                pltpu.make_async_copy(mo, acc.at[cia.at[1 - b]], sco).wait()

            @pl.when(q + 1 < NCHUNK)
            def _():
                pltpu.async_copy(msg.at[pl.ds(base + (q + 1) * C, C)], mo, go)
                pltpu.async_copy(cent2.at[wid, pl.ds((q + 1) * C, C)],
                                 cia.at[1 - b], cio)

            pltpu.make_async_copy(msg.at[pl.ds(base + q * C, C)], m, g).wait()

            @pl.when(q >= 1)
            def _():
                pltpu.make_async_copy(cent2.at[wid, pl.ds(q * C, C)],
                                      cia.at[b], ci).wait()

            pltpu.async_copy(m, acc.at[cia.at[b]], sc, add=True)
        return carry

    lax.fori_loop(0, NCHUNK // 2, body, 0)
    pltpu.make_async_copy(m1, acc.at[cia.at[1]], sc1).wait()
    plsc.subcore_barrier()
    pltpu.sync_copy(acc.at[pl.ds(r0, ROWS_PER_TILE)],
                    out.at[cid, pl.ds(r0, ROWS_PER_TILE)])


_scatter = functools.partial(
    pl.kernel,
    out_type=jax.ShapeDtypeStruct((2, N_ACC, ATOM_DIM), jnp.float32),
    mesh=_mesh,
    scratch_types=[
        pltpu.VMEM((2, C), jnp.int32),
        pltpu.VMEM((C, ATOM_DIM), jnp.float32),
        pltpu.VMEM((C, ATOM_DIM), jnp.float32),
        pltpu.VMEM_SHARED((N_ACC, ATOM_DIM), jnp.float32),
        pltpu.SemaphoreType.DMA,
        pltpu.SemaphoreType.DMA,
        pltpu.SemaphoreType.DMA,
        pltpu.SemaphoreType.DMA,
        pltpu.SemaphoreType.DMA,
        pltpu.SemaphoreType.DMA,
    ],
)(_scatter_body)


def _final_body(p0_ref, p1_ref, wout_ref, bout_ref, af_ref, o_ref):
    a = p0_ref[...] + p1_ref[...]
    o_ref[...] = (
        jnp.dot(a, wout_ref[...], preferred_element_type=jnp.float32)
        + bout_ref[...]
        + af_ref[...]
    )


def kernel(atom_feas, bond_feas, bond_weights, atom_graph, directed2undirected,
           W1c, b1c, W2c, b2c, W1g, b1g, W2g, b2g, Wout, bout):
    f32 = jnp.float32
    Wctr = jnp.concatenate([W1c[:ATOM_DIM], W1g[:ATOM_DIM]], axis=1)
    Wnbr = jnp.concatenate([W1c[ATOM_DIM + 16:], W1g[ATOM_DIM + 16:]], axis=1)
    Wbnd = jnp.concatenate([W1c[ATOM_DIM:ATOM_DIM + 16],
                            W1g[ATOM_DIM:ATOM_DIM + 16]], axis=1)
    bcat1 = jnp.concatenate([b1c, b1g])[None, :]
    W2blk = jnp.zeros((ATOM_DIM, 2 * ATOM_DIM), f32)
    W2blk = W2blk.at[:HIDDEN, :ATOM_DIM].set(W2c)
    W2blk = W2blk.at[HIDDEN:, ATOM_DIM:].set(W2g)
    bcat2 = jnp.concatenate([b2c, b2g])[None, :]

    pad = N_PAD - N_DIR
    spread = jnp.arange(pad, dtype=jnp.int32)
    cent = jnp.concatenate(
        [atom_graph[:, 0], jnp.full((pad,), N_ATOMS, jnp.int32)])
    centg = jnp.concatenate([atom_graph[:, 0], spread % N_ATOMS])
    nbrs = jnp.concatenate([atom_graph[:, 1], spread % N_ATOMS])
    und = jnp.concatenate([directed2undirected, spread % N_UND])
    cent2 = cent.reshape(NW, E_W)
    centg2 = centg.reshape(NW, E_W)
    nbr2 = nbrs.reshape(NW, E_W)
    und2 = und.reshape(NW, E_W)
    af_pad = jnp.concatenate(
        [atom_feas, jnp.zeros((N_ACC - N_ATOMS, ATOM_DIM), f32)])

    pctr, pnbr = pl.pallas_call(
        _ptables_body,
        out_shape=[
            jax.ShapeDtypeStruct((N_ACC, ATOM_DIM), f32),
            jax.ShapeDtypeStruct((N_ACC, ATOM_DIM), f32),
        ],
    )(af_pad, Wctr, Wnbr)

    btab = pl.pallas_call(
        _btable_body,
        grid=(20,),
        in_specs=[
            pl.BlockSpec((N_UND // 20, 16), lambda i: (i, 0)),
            pl.BlockSpec((N_UND // 20, ATOM_DIM), lambda i: (i, 0)),
            pl.BlockSpec((16, ATOM_DIM), lambda i: (0, 0)),
            pl.BlockSpec((1, ATOM_DIM), lambda i: (0, 0)),
        ],
        out_specs=pl.BlockSpec((N_UND // 20, ATOM_DIM), lambda i: (i, 0)),
        out_shape=jax.ShapeDtypeStruct((N_UND, ATOM_DIM), jnp.int32),
    )(bond_feas, bond_weights, Wbnd, bcat1)

    hp, bslab = _gather_h(pctr, pnbr, btab, centg2, nbr2, und2)

    BLK = 4096
    msg = pl.pallas_call(
        _mlp_body,
        grid=(N_PAD // BLK,),
        in_specs=[
            pl.BlockSpec((BLK, ATOM_DIM), lambda i: (i, 0)),
            pl.BlockSpec((BLK, ATOM_DIM), lambda i: (i, 0)),
            pl.BlockSpec((ATOM_DIM, 2 * ATOM_DIM), lambda i: (0, 0)),
            pl.BlockSpec((1, 2 * ATOM_DIM), lambda i: (0, 0)),
        ],
        out_specs=pl.BlockSpec((BLK, ATOM_DIM), lambda i: (i, 0)),
        out_shape=jax.ShapeDtypeStruct((N_PAD, ATOM_DIM), f32),
    )(hp, bslab, W2blk, bcat2)

    zeros = jnp.zeros((N_ACC, ATOM_DIM), f32)
    partials = _scatter(msg, cent2, zeros)

    out = pl.pallas_call(
        _final_body,
        out_shape=jax.ShapeDtypeStruct((N_ATOMS, ATOM_DIM), f32),
    )(partials[0, :N_ATOMS], partials[1, :N_ATOMS], Wout, bout[None, :],
      atom_feas)
    return out


# speedup vs baseline: 1.9401x; 1.0494x over previous
import functools

import jax
import jax.numpy as jnp
from jax import lax
from jax.experimental import pallas as pl
from jax.experimental.pallas import tpu as pltpu
from jax.experimental.pallas import tpu_sc as plsc

N_ATOMS = 10000
N_DIR = 320000
N_UND = 160000
ATOM_DIM = 128
HIDDEN = 64

NW = 32
C = 64
NCHUNK = 158
E_W = NCHUNK * C
N_PAD = NW * E_W
N_HALF = N_PAD // 2
E_H = N_HALF // NW
NCH = E_H // C
N_ACC = 10112
ROWS_PER_TILE = N_ACC // 16

_mesh = plsc.VectorSubcoreMesh(core_axis_name="c", subcore_axis_name="s")
_HI = -65536


def _ptables_body(af_ref, wctr_ref, wnbr_ref, pc_ref, pn_ref):
    af = af_ref[...]
    pc_ref[...] = jnp.dot(af, wctr_ref[...], preferred_element_type=jnp.float32)
    pn_ref[...] = jnp.dot(af, wnbr_ref[...], preferred_element_type=jnp.float32)


def _btable_body(bf_ref, bw_ref, wb_ref, bb_ref, out_ref):
    proj = (jnp.dot(bf_ref[...], wb_ref[...],
                    preferred_element_type=jnp.float32) + bb_ref[...])
    lo = lax.bitcast_convert_type(proj.astype(jnp.bfloat16), jnp.uint16
                                  ).astype(jnp.int32)
    hi = lax.bitcast_convert_type(bw_ref[...].astype(jnp.bfloat16), jnp.uint16
                                  ).astype(jnp.int32)
    out_ref[...] = jnp.bitwise_or(lo, jnp.left_shift(hi, 16))


def _make_gather(off):
    def body(pctr, pnbr, btab, centg, nbrsf, undf, hp, bslab,
             cia, nia, uia,
             bc0, bn0, bb0, o0, bc1, bn1, bb1, o1,
             g0, g1, s0, s1):
        wid = lax.axis_index("s") * 2 + lax.axis_index("c")
        base = wid * E_H
        pltpu.sync_copy(centg.at[pl.ds(off + base, E_H)], cia)
        pltpu.sync_copy(nbrsf.at[pl.ds(off + base, E_H)], nia)
        pltpu.sync_copy(undf.at[pl.ds(off + base, E_H)], uia)
        sets = ((bc0, bn0, bb0, o0, g0, s0), (bc1, bn1, bb1, o1, g1, s1))

        def fire(q, st):
            bc, bn, bb, _, g, _ = st
            pltpu.async_copy(pctr.at[cia.at[pl.ds(q * C, C)]], bc, g)
            pltpu.async_copy(pnbr.at[nia.at[pl.ds(q * C, C)]], bn, g)
            pltpu.async_copy(btab.at[uia.at[pl.ds(q * C, C)]], bb, g)

        def wait_gathers(q, st):
            bc, bn, bb, _, g, _ = st
            pltpu.make_async_copy(pctr.at[cia.at[pl.ds(q * C, C)]], bc, g).wait()
            pltpu.make_async_copy(pnbr.at[nia.at[pl.ds(q * C, C)]], bn, g).wait()
            pltpu.make_async_copy(btab.at[uia.at[pl.ds(q * C, C)]], bb, g).wait()

        def fire_stores(q, st):
            bc, bn, bb, o, _, s = st
            sl = pl.ds(base + q * C, C)
            pltpu.async_copy(o, hp.at[sl], s)
            pltpu.async_copy(bb, bslab.at[sl], s)

        def wait_stores(q, st):
            bc, bn, bb, o, _, s = st
            sl = pl.ds(base + q * C, C)
            pltpu.make_async_copy(o, hp.at[sl], s).wait()
            pltpu.make_async_copy(bb, bslab.at[sl], s).wait()

        def compute(st):
            bc, bn, bb, o, _, _ = st

            def row(r, rc):
                for j in range(ATOM_DIM // 16):
                    sl = (r, pl.ds(j * 16, 16))
                    o[sl] = bc[sl] + bn[sl]
                return rc

            lax.fori_loop(0, C, row, 0)

        fire(0, sets[0])

        def loop(k, carry):
            for b in (0, 1):
                q = 2 * k + b
                st = sets[b]

                @pl.when(q >= 1)
                def _():
                    wait_stores(q - 1, sets[1 - b])

                @pl.when(q + 1 < NCH)
                def _():
                    fire(q + 1, sets[1 - b])

                wait_gathers(q, st)
                compute(st)
                fire_stores(q, st)
            return carry

        lax.fori_loop(0, NCH // 2, loop, 0)
        wait_stores(NCH - 2, sets[1])
        wait_gathers(NCH - 1, sets[0])
        compute(sets[0])
        fire_stores(NCH - 1, sets[0])
        wait_stores(NCH - 1, sets[0])

    return functools.partial(
        pl.kernel,
        out_type=[
            jax.ShapeDtypeStruct((N_HALF, ATOM_DIM), jnp.float32),
            jax.ShapeDtypeStruct((N_HALF, ATOM_DIM), jnp.int32),
        ],
        mesh=_mesh,
        scratch_types=[
            pltpu.VMEM((E_H,), jnp.int32),
            pltpu.VMEM((E_H,), jnp.int32),
            pltpu.VMEM((E_H,), jnp.int32),
            pltpu.VMEM((C, ATOM_DIM), jnp.float32),
            pltpu.VMEM((C, ATOM_DIM), jnp.float32),
            pltpu.VMEM((C, ATOM_DIM), jnp.int32),
            pltpu.VMEM((C, ATOM_DIM), jnp.float32),
            pltpu.VMEM((C, ATOM_DIM), jnp.float32),
            pltpu.VMEM((C, ATOM_DIM), jnp.float32),
            pltpu.VMEM((C, ATOM_DIM), jnp.int32),
            pltpu.VMEM((C, ATOM_DIM), jnp.float32),
            pltpu.SemaphoreType.DMA,
            pltpu.SemaphoreType.DMA,
            pltpu.SemaphoreType.DMA,
            pltpu.SemaphoreType.DMA,
        ],
    )(body)


_gather_a = _make_gather(0)
_gather_b = _make_gather(N_HALF)


def _mlp_body(hp_ref, bs_ref, w2_ref, b2_ref, o_ref):
    u = bs_ref[...]
    h = hp_ref[...] + lax.bitcast_convert_type(
        jnp.left_shift(u, 16), jnp.float32)
    bw = lax.bitcast_convert_type(jnp.bitwise_and(u, _HI), jnp.float32)
    s = h * jax.nn.sigmoid(h)
    t = jnp.dot(s, w2_ref[...], preferred_element_type=jnp.float32) + b2_ref[...]
    core = t[:, :ATOM_DIM]
    gate = t[:, ATOM_DIM:]
    o_ref[...] = core * jax.nn.sigmoid(core) * jax.nn.sigmoid(gate) * bw


def _make_scatter(off):
    def body(msg, centf, zeros, out,
             cia, m0, m1, acc,
             g0, g1, sc0, sc1, ci0, ci1):
        cid = lax.axis_index("c")
        sid = lax.axis_index("s")
        wid = sid * 2 + cid
        r0 = sid * ROWS_PER_TILE
        pltpu.sync_copy(zeros.at[pl.ds(r0, ROWS_PER_TILE)],
                        acc.at[pl.ds(r0, ROWS_PER_TILE)])
        plsc.subcore_barrier()
        base = wid * E_H
        pltpu.sync_copy(centf.at[pl.ds(off + base, C)], cia.at[0])
        sets = ((m0, g0, sc0, ci0), (m1, g1, sc1, ci1))

        pltpu.async_copy(msg.at[pl.ds(base, C)], m0, g0)

        def loop(k, carry):
            for b in (0, 1):
                q = 2 * k + b
                m, g, sc, ci = sets[b]
                mo, go, sco, cio = sets[1 - b]

                @pl.when(q >= 1)
                def _():
                    pltpu.make_async_copy(mo, acc.at[cia.at[1 - b]], sco).wait()

                @pl.when(q + 1 < NCH)
                def _():
                    pltpu.async_copy(msg.at[pl.ds(base + (q + 1) * C, C)],
                                     mo, go)
                    pltpu.async_copy(
                        centf.at[pl.ds(off + base + (q + 1) * C, C)],
                        cia.at[1 - b], cio)

                pltpu.make_async_copy(msg.at[pl.ds(base + q * C, C)], m,
                                      g).wait()

                @pl.when(q >= 1)
                def _():
                    pltpu.make_async_copy(
                        centf.at[pl.ds(off + base + q * C, C)],
                        cia.at[b], ci).wait()

                pltpu.async_copy(m, acc.at[cia.at[b]], sc, add=True)
            return carry

        lax.fori_loop(0, NCH // 2, loop, 0)
        q = NCH - 1
        pltpu.make_async_copy(m1, acc.at[cia.at[1]], sc1).wait()
        pltpu.make_async_copy(msg.at[pl.ds(base + q * C, C)], m0, g0).wait()
        pltpu.make_async_copy(centf.at[pl.ds(off + base + q * C, C)],
                              cia.at[0], ci0).wait()
        pltpu.async_copy(m0, acc.at[cia.at[0]], sc0, add=True)
        pltpu.make_async_copy(m0, acc.at[cia.at[0]], sc0).wait()
        plsc.subcore_barrier()
        pltpu.sync_copy(acc.at[pl.ds(r0, ROWS_PER_TILE)],
                        out.at[cid, pl.ds(r0, ROWS_PER_TILE)])

    return functools.partial(
        pl.kernel,
        out_type=jax.ShapeDtypeStruct((2, N_ACC, ATOM_DIM), jnp.float32),
        mesh=_mesh,
        scratch_types=[
            pltpu.VMEM((2, C), jnp.int32),
            pltpu.VMEM((C, ATOM_DIM), jnp.float32),
            pltpu.VMEM((C, ATOM_DIM), jnp.float32),
            pltpu.VMEM_SHARED((N_ACC, ATOM_DIM), jnp.float32),
            pltpu.SemaphoreType.DMA,
            pltpu.SemaphoreType.DMA,
            pltpu.SemaphoreType.DMA,
            pltpu.SemaphoreType.DMA,
            pltpu.SemaphoreType.DMA,
            pltpu.SemaphoreType.DMA,
        ],
    )(body)


_scatter_a = _make_scatter(0)
_scatter_b = _make_scatter(N_HALF)


def _final_body(pa_ref, pb_ref, wout_ref, bout_ref, af_ref, o_ref):
    a = (pa_ref[0] + pa_ref[1]) + (pb_ref[0] + pb_ref[1])
    o_ref[...] = (
        jnp.dot(a, wout_ref[...], preferred_element_type=jnp.float32)
        + bout_ref[...]
        + af_ref[...]
    )


def kernel(atom_feas, bond_feas, bond_weights, atom_graph, directed2undirected,
           W1c, b1c, W2c, b2c, W1g, b1g, W2g, b2g, Wout, bout):
    f32 = jnp.float32
    Wctr = jnp.concatenate([W1c[:ATOM_DIM], W1g[:ATOM_DIM]], axis=1)
    Wnbr = jnp.concatenate([W1c[ATOM_DIM + 16:], W1g[ATOM_DIM + 16:]], axis=1)
    Wbnd = jnp.concatenate([W1c[ATOM_DIM:ATOM_DIM + 16],
                            W1g[ATOM_DIM:ATOM_DIM + 16]], axis=1)
    bcat1 = jnp.concatenate([b1c, b1g])[None, :]
    W2blk = jnp.zeros((ATOM_DIM, 2 * ATOM_DIM), f32)
    W2blk = W2blk.at[:HIDDEN, :ATOM_DIM].set(W2c)
    W2blk = W2blk.at[HIDDEN:, ATOM_DIM:].set(W2g)
    bcat2 = jnp.concatenate([b2c, b2g])[None, :]

    pad = N_PAD - N_DIR
    spread = jnp.arange(pad, dtype=jnp.int32)
    cent = jnp.concatenate(
        [atom_graph[:, 0], jnp.full((pad,), N_ATOMS, jnp.int32)])
    centg = jnp.concatenate([atom_graph[:, 0], spread % N_ATOMS])
    nbrs = jnp.concatenate([atom_graph[:, 1], spread % N_ATOMS])
    und = jnp.concatenate([directed2undirected, spread % N_UND])
    af_pad = jnp.concatenate(
        [atom_feas, jnp.zeros((N_ACC - N_ATOMS, ATOM_DIM), f32)])

    pctr, pnbr = pl.pallas_call(
        _ptables_body,
        out_shape=[
            jax.ShapeDtypeStruct((N_ACC, ATOM_DIM), f32),
            jax.ShapeDtypeStruct((N_ACC, ATOM_DIM), f32),
        ],
    )(af_pad, Wctr, Wnbr)

    btab = pl.pallas_call(
        _btable_body,
        grid=(20,),
        in_specs=[
            pl.BlockSpec((N_UND // 20, 16), lambda i: (i, 0)),
            pl.BlockSpec((N_UND // 20, ATOM_DIM), lambda i: (i, 0)),
            pl.BlockSpec((16, ATOM_DIM), lambda i: (0, 0)),
            pl.BlockSpec((1, ATOM_DIM), lambda i: (0, 0)),
        ],
        out_specs=pl.BlockSpec((N_UND // 20, ATOM_DIM), lambda i: (i, 0)),
        out_shape=jax.ShapeDtypeStruct((N_UND, ATOM_DIM), jnp.int32),
    )(bond_feas, bond_weights, Wbnd, bcat1)

    BLK = 2048
    zeros = jnp.zeros((N_ACC, ATOM_DIM), f32)

    def mlp_half(hp, bslab):
        return pl.pallas_call(
            _mlp_body,
            grid=(N_HALF // BLK,),
            in_specs=[
                pl.BlockSpec((BLK, ATOM_DIM), lambda i: (i, 0)),
                pl.BlockSpec((BLK, ATOM_DIM), lambda i: (i, 0)),
                pl.BlockSpec((ATOM_DIM, 2 * ATOM_DIM), lambda i: (0, 0)),
                pl.BlockSpec((1, 2 * ATOM_DIM), lambda i: (0, 0)),
            ],
            out_specs=pl.BlockSpec((BLK, ATOM_DIM), lambda i: (i, 0)),
            out_shape=jax.ShapeDtypeStruct((N_HALF, ATOM_DIM), f32),
        )(hp, bslab, W2blk, bcat2)

    hp_a, bs_a = _gather_a(pctr, pnbr, btab, centg, nbrs, und)
    hp_b, bs_b = _gather_b(pctr, pnbr, btab, centg, nbrs, und)
    msg_a = mlp_half(hp_a, bs_a)
    msg_b = mlp_half(hp_b, bs_b)
    part_a = _scatter_a(msg_a, cent, zeros)
    part_b = _scatter_b(msg_b, cent, zeros)

    out = pl.pallas_call(
        _final_body,
        out_shape=jax.ShapeDtypeStruct((N_ATOMS, ATOM_DIM), f32),
    )(part_a[:, :N_ATOMS], part_b[:, :N_ATOMS], Wout, bout[None, :],
      atom_feas)
    return out
